# Initial kernel scaffold; baseline (speedup 1.0000x reference)
import jax
import jax.numpy as jnp
from jax.experimental import pallas as pl


def kernel(student_feats, teacher_feats, targets, num_old_classes):
    raise NotImplementedError("write your pallas kernel here")


# speedup vs baseline: 1.0438x; 1.0438x over previous
import functools

import jax
import jax.numpy as jnp
from jax.experimental import pallas as pl

_FEATURE_DIM = 512
_TOP_K = 50
_ALPHA = 0.01
_TEMPERATURE = 0.1
_OLD_W = 1.0
_NEW_W = 0.3


def _topk_mask(x):
    m = x.shape[0]
    bits = jax.lax.bitcast_convert_type(x, jnp.int32)
    v0 = (bits & jnp.int32(0x7FFFFFFF)) << 1
    active0 = jnp.ones(x.shape, jnp.float32)
    in0 = jnp.zeros(x.shape, jnp.float32)
    k0 = jnp.full((m, 1), float(_TOP_K), jnp.float32)

    def body(_, carry):
        v, active, inm, k = carry
        hi = active * jnp.where(v < 0, 1.0, 0.0)
        cnt = jnp.sum(hi, axis=1, keepdims=True)
        cond = cnt >= k
        active = jnp.where(cond, hi, active - hi)
        inm = jnp.where(cond, inm, inm + hi)
        k = jnp.where(cond, k, k - cnt)
        return (v << 1, active, inm, k)

    _, active, inm, _ = jax.lax.fori_loop(0, 31, body, (v0, active0, in0, k0))
    return inm + active


def _body(s_ref, t_ref, w_ref, num_ref, den_ref):
    i = pl.program_id(0)
    s = s_ref[...]
    t = t_ref[...]
    r = s.shape[0]
    x = jnp.concatenate([s, t], axis=0)
    mask = _topk_mask(x)
    xm = x * (_ALPHA + (1.0 - _ALPHA) * mask)
    sm = xm[:r]
    tm = xm[r:]
    dot = jnp.sum(sm * tm, axis=1, keepdims=True)
    ss = jnp.sum(sm * sm, axis=1, keepdims=True)
    tt = jnp.sum(tm * tm, axis=1, keepdims=True)
    cos = dot / ((jnp.sqrt(ss) + 1e-8) * (jnp.sqrt(tt) + 1e-8))
    per = (1.0 - cos) / _TEMPERATURE
    w = w_ref[0]
    pnum = jnp.dot(w, per, preferred_element_type=jnp.float32)
    pden = jnp.sum(w, axis=1, keepdims=True)

    @pl.when(i == 0)
    def _():
        num_ref[...] = pnum
        den_ref[...] = pden

    @pl.when(i > 0)
    def _():
        num_ref[...] += pnum
        den_ref[...] += pden


@functools.partial(jax.jit, static_argnames=())
def kernel(student_feats, teacher_feats, targets, num_old_classes):
    b, d = student_feats.shape
    r = 256
    g = b // r
    w = jnp.where(targets < num_old_classes, _OLD_W, _NEW_W).astype(jnp.float32)
    w3 = w.reshape(g, 1, r)
    num, den = pl.pallas_call(
        _body,
        grid=(g,),
        in_specs=[
            pl.BlockSpec((r, d), lambda i: (i, 0)),
            pl.BlockSpec((r, d), lambda i: (i, 0)),
            pl.BlockSpec((1, 1, r), lambda i: (i, 0, 0)),
        ],
        out_specs=[
            pl.BlockSpec((1, 1), lambda i: (0, 0)),
            pl.BlockSpec((1, 1), lambda i: (0, 0)),
        ],
        out_shape=[
            jax.ShapeDtypeStruct((1, 1), jnp.float32),
            jax.ShapeDtypeStruct((1, 1), jnp.float32),
        ],
    )(student_feats, teacher_feats, w3)
    return (num[0, 0] / (den[0, 0] + 1e-8)).astype(jnp.float32)


# speedup vs baseline: 1.3169x; 1.2617x over previous
import functools

import jax
import jax.numpy as jnp
from jax.experimental import pallas as pl

_FEATURE_DIM = 512
_TOP_K = 50
_ALPHA = 0.01
_TEMPERATURE = 0.1
_OLD_W = 1.0
_NEW_W = 0.3


def _topk_factor(x):
    m = x.shape[0]
    bits = jax.lax.bitcast_convert_type(x, jnp.int32)
    av = bits & jnp.int32(0x7FFFFFFF)
    v0 = av << 1
    active0 = jnp.full(x.shape, -1, jnp.int32)
    nk0 = jnp.full((m, 1), -_TOP_K, jnp.int32)
    p0 = jnp.zeros((m, 1), jnp.int32)

    def body(i, carry):
        v, active, nk, p = carry
        hi = active & (v >> 31)
        cnt = jnp.sum(hi, axis=1, keepdims=True)
        cond = cnt <= nk
        bit = jnp.int32(1) << (30 - i)
        active = jnp.where(cond, hi, active ^ hi)
        p = jnp.where(cond, p | bit, p)
        nk = jnp.where(cond, nk, nk - cnt)
        return (v << 1, active, nk, p)

    _, _, _, p = jax.lax.fori_loop(0, 31, body, (v0, active0, nk0, p0))
    return jnp.where(av >= p, 1.0, _ALPHA)


def _body(s_ref, t_ref, w_ref, num_ref, den_ref):
    i = pl.program_id(0)
    s = s_ref[...]
    t = t_ref[...]
    r = s.shape[0]
    x = jnp.concatenate([s, t], axis=0)
    xm = x * _topk_factor(x)
    sm = xm[:r]
    tm = xm[r:]
    dot = jnp.sum(sm * tm, axis=1, keepdims=True)
    ss = jnp.sum(sm * sm, axis=1, keepdims=True)
    tt = jnp.sum(tm * tm, axis=1, keepdims=True)
    cos = dot / ((jnp.sqrt(ss) + 1e-8) * (jnp.sqrt(tt) + 1e-8))
    per = (1.0 - cos) / _TEMPERATURE
    w = w_ref[0]
    pnum = jnp.dot(w, per, preferred_element_type=jnp.float32)
    pden = jnp.sum(w, axis=1, keepdims=True)

    @pl.when(i == 0)
    def _():
        num_ref[...] = pnum
        den_ref[...] = pden

    @pl.when(i > 0)
    def _():
        num_ref[...] += pnum
        den_ref[...] += pden


@functools.partial(jax.jit, static_argnames=())
def kernel(student_feats, teacher_feats, targets, num_old_classes):
    b, d = student_feats.shape
    r = 256
    g = b // r
    w = jnp.where(targets < num_old_classes, _OLD_W, _NEW_W).astype(jnp.float32)
    w3 = w.reshape(g, 1, r)
    num, den = pl.pallas_call(
        _body,
        grid=(g,),
        in_specs=[
            pl.BlockSpec((r, d), lambda i: (i, 0)),
            pl.BlockSpec((r, d), lambda i: (i, 0)),
            pl.BlockSpec((1, 1, r), lambda i: (i, 0, 0)),
        ],
        out_specs=[
            pl.BlockSpec((1, 1), lambda i: (0, 0)),
            pl.BlockSpec((1, 1), lambda i: (0, 0)),
        ],
        out_shape=[
            jax.ShapeDtypeStruct((1, 1), jnp.float32),
            jax.ShapeDtypeStruct((1, 1), jnp.float32),
        ],
    )(student_feats, teacher_feats, w3)
    return (num[0, 0] / (den[0, 0] + 1e-8)).astype(jnp.float32)


# speedup vs baseline: 10.2858x; 7.8103x over previous
import functools

import jax
import jax.numpy as jnp
from jax import lax
from jax.experimental import pallas as pl
from jax.experimental.pallas import tpu as pltpu
from jax.experimental.pallas import tpu_sc as plsc

_FEATURE_DIM = 512
_TOP_K = 50
_ALPHA = 0.01
_TEMPERATURE = 0.1
_OLD_W = 1.0
_NEW_W = 0.3

_NC = 2
_NS = 16
_NW = _NC * _NS
_LANES = 16
_CHUNK = 64


def _vsort(x):
    k, _ = plsc.sort_key_val(x, x)
    return k


def _sort_bitonic(x):
    m = len(x)
    if m == 1:
        return [_vsort(x[0])]
    half = m // 2
    lo = [jnp.minimum(x[j], x[j + half]) for j in range(half)]
    hi = [jnp.maximum(x[j], x[j + half]) for j in range(half)]
    return _sort_bitonic(lo) + _sort_bitonic(hi)


def _merge_full(a, b):
    m = len(a)
    rev_b = [jnp.flip(v, 0) for v in reversed(b)]
    lo = [jnp.minimum(a[j], rev_b[j]) for j in range(m)]
    hi = [jnp.maximum(a[j], rev_b[j]) for j in range(m)]
    return _sort_bitonic(lo) + _sort_bitonic(hi)


def _merge_top(a, b):
    m = len(a)
    rev_b = [jnp.flip(v, 0) for v in reversed(b)]
    hi = [jnp.maximum(a[j], rev_b[j]) for j in range(m)]
    return _sort_bitonic(hi)


def _row_threshold(buf, i):
    lvl = [[_vsort(jnp.abs(buf[i, pl.ds(_LANES * q, _LANES)]))]
           for q in range(_FEATURE_DIM // _LANES)]
    lvl = [_merge_full(lvl[2 * a], lvl[2 * a + 1]) for a in range(16)]
    lvl = [_merge_full(lvl[2 * a], lvl[2 * a + 1]) for a in range(8)]
    lvl = [_merge_top(lvl[2 * a], lvl[2 * a + 1]) for a in range(4)]
    lvl = [_merge_top(lvl[2 * a], lvl[2 * a + 1]) for a in range(2)]
    top64 = _merge_top(lvl[0], lvl[1])
    lane = lax.iota(jnp.int32, _LANES)
    return jnp.sum(jnp.where(lane == 14, top64[0], 0.0))


def _sc_body(s_hbm, t_hbm, ths_hbm, tht_hbm, buf, thr_buf):
    wid = lax.axis_index("s") * _NC + lax.axis_index("c")
    rows_per_w = s_hbm.shape[0] // _NW
    base = wid * rows_per_w
    lane = lax.iota(jnp.int32, _LANES)

    for src, dst in ((s_hbm, ths_hbm), (t_hbm, tht_hbm)):

        def chunk_body(c, _, src=src, dst=dst):
            off = pl.multiple_of(base + c * _CHUNK, _CHUNK)
            pltpu.sync_copy(src.at[pl.ds(off, _CHUNK)], buf)

            def group_body(g, _):
                def row_body(j, acc):
                    th = _row_threshold(buf, g * _LANES + j)
                    return jnp.where(lane == j, th, acc)

                acc = lax.fori_loop(0, _LANES, row_body,
                                    jnp.zeros((_LANES,), jnp.float32))
                goff = pl.multiple_of(g * _LANES, _LANES)
                thr_buf[pl.ds(goff, _LANES)] = acc
                return 0

            lax.fori_loop(0, _CHUNK // _LANES, group_body, 0)
            pltpu.sync_copy(thr_buf, dst.at[pl.ds(off, _CHUNK)])
            return 0

        lax.fori_loop(0, rows_per_w // _CHUNK, chunk_body, 0)


def _sc_thresholds(student_feats, teacher_feats):
    b = student_feats.shape[0]
    mesh = plsc.VectorSubcoreMesh(core_axis_name="c", subcore_axis_name="s")
    fn = functools.partial(
        pl.kernel,
        mesh=mesh,
        out_type=[
            jax.ShapeDtypeStruct((b,), jnp.float32),
            jax.ShapeDtypeStruct((b,), jnp.float32),
        ],
        scratch_types=[
            pltpu.VMEM((_CHUNK, _FEATURE_DIM), jnp.float32),
            pltpu.VMEM((_CHUNK,), jnp.float32),
        ],
        compiler_params=pltpu.CompilerParams(needs_layout_passes=False),
    )(_sc_body)
    return fn(student_feats, teacher_feats)


def _tc_body(s_ref, t_ref, ths_ref, tht_ref, w_ref, num_ref, den_ref):
    i = pl.program_id(0)
    s = s_ref[...]
    t = t_ref[...]
    fs = jnp.where(jnp.abs(s) >= ths_ref[...], 1.0, _ALPHA)
    ft = jnp.where(jnp.abs(t) >= tht_ref[...], 1.0, _ALPHA)
    sm = s * fs
    tm = t * ft
    dot = jnp.sum(sm * tm, axis=1, keepdims=True)
    ss = jnp.sum(sm * sm, axis=1, keepdims=True)
    tt = jnp.sum(tm * tm, axis=1, keepdims=True)
    cos = dot / ((jnp.sqrt(ss) + 1e-8) * (jnp.sqrt(tt) + 1e-8))
    per = (1.0 - cos) / _TEMPERATURE
    w = w_ref[0]
    pnum = jnp.dot(w, per, preferred_element_type=jnp.float32)
    pden = jnp.sum(w, axis=1, keepdims=True)

    @pl.when(i == 0)
    def _():
        num_ref[...] = pnum
        den_ref[...] = pden

    @pl.when(i > 0)
    def _():
        num_ref[...] += pnum
        den_ref[...] += pden


def kernel(student_feats, teacher_feats, targets, num_old_classes):
    b, d = student_feats.shape
    ths, tht = _sc_thresholds(student_feats, teacher_feats)
    r = 512
    g = b // r
    w = jnp.where(targets < num_old_classes, _OLD_W, _NEW_W).astype(jnp.float32)
    num, den = pl.pallas_call(
        _tc_body,
        grid=(g,),
        in_specs=[
            pl.BlockSpec((r, d), lambda i: (i, 0)),
            pl.BlockSpec((r, d), lambda i: (i, 0)),
            pl.BlockSpec((r, 1), lambda i: (i, 0)),
            pl.BlockSpec((r, 1), lambda i: (i, 0)),
            pl.BlockSpec((1, 1, r), lambda i: (i, 0, 0)),
        ],
        out_specs=[
            pl.BlockSpec((1, 1), lambda i: (0, 0)),
            pl.BlockSpec((1, 1), lambda i: (0, 0)),
        ],
        out_shape=[
            jax.ShapeDtypeStruct((1, 1), jnp.float32),
            jax.ShapeDtypeStruct((1, 1), jnp.float32),
        ],
    )(student_feats, teacher_feats, ths.reshape(b, 1), tht.reshape(b, 1),
      w.reshape(g, 1, r))
    return (num[0, 0] / (den[0, 0] + 1e-8)).astype(jnp.float32)
